# baseline (speedup 1.0000x reference)
import functools

import jax
import jax.numpy as jnp
from jax import lax
from jax.experimental import pallas as pl
from jax.experimental.pallas import tpu as pltpu
from jax.experimental.pallas import tpu_sc as plsc

N = 10000
D = 128
L = 5

NC = 2
NS = 16
NW = NC * NS
CHUNK = 128
N_ACC = 10240
ROWS_PER_SUB = N_ACC // NS
HW = D // NC
CW = 32
NBUF = 2
SPC = 2
RPS = SPC * CHUNK


def _sc_segment_sum(table, gidx, didx, nch, width, partition):
    mesh = plsc.VectorSubcoreMesh(core_axis_name="c", subcore_axis_name="s")
    zrows = 64
    nfull = ROWS_PER_SUB // zrows

    @functools.partial(
        pl.kernel,
        mesh=mesh,
        compiler_params=pltpu.CompilerParams(use_tc_tiling_on_sc=False),
        out_type=jax.ShapeDtypeStruct((NC, N_ACC, width), jnp.float32),
        scratch_types=[
            pltpu.VMEM((nch + NBUF, RPS), jnp.int32),
            pltpu.VMEM((nch + NBUF, RPS), jnp.int32),
            pltpu.VMEM((NBUF, RPS, width), jnp.float32),
            pltpu.VMEM((zrows, width), jnp.float32),
            pltpu.VMEM_SHARED((N_ACC, width), jnp.float32),
        ] + [pltpu.SemaphoreType.DMA] * (2 * NBUF),
    )
    def k(table_hbm, gidx_hbm, didx_hbm, out_hbm,
          g_v, d_v, buf, zbuf, acc, *sems):
        semg = sems[:NBUF]
        c = lax.axis_index("c")
        s = lax.axis_index("s")
        w = c * NS + s if partition else s
        pltpu.sync_copy(gidx_hbm.at[w], g_v)
        pltpu.sync_copy(didx_hbm.at[w], d_v)

        def zrow(i, carry):
            for col in range(width // 16):
                zbuf[i, pl.ds(col * 16, 16)] = jnp.zeros((16,), jnp.float32)
            return carry
        lax.fori_loop(0, zrows, zrow, 0)

        base = s * ROWS_PER_SUB
        for kk in range(nfull):
            pltpu.sync_copy(zbuf, acc.at[pl.ds(base + kk * zrows, zrows)])
        plsc.subcore_barrier()

        tbl = table_hbm.at[c]
        for b in range(NBUF):
            pltpu.async_copy(tbl.at[g_v.at[b]], buf.at[b], semg[b])

        def body(i, carry):
            j0 = NBUF * i
            for b in range(NBUF):
                pltpu.make_async_copy(tbl.at[g_v.at[j0 + b]], buf.at[b],
                                      semg[b]).wait()
                pltpu.sync_copy(buf.at[b], acc.at[d_v.at[j0 + b]], add=True)
                pltpu.async_copy(tbl.at[g_v.at[j0 + NBUF + b]], buf.at[b],
                                 semg[b])
            return carry
        lax.fori_loop(0, nch // NBUF, body, 0)

        for b in range(NBUF):
            pltpu.make_async_copy(tbl.at[g_v.at[nch + b]], buf.at[b],
                                  semg[b]).wait()

        plsc.subcore_barrier()
        pltpu.sync_copy(acc.at[pl.ds(base, ROWS_PER_SUB)],
                        out_hbm.at[c, pl.ds(base, ROWS_PER_SUB)])

    return k(table, gidx, didx)


def _epilogue_body(relu, p_ref, ht_ref, cnts_ref, etab_ref, gamma_ref,
                   beta_ref, out_ref):
    cm = (cnts_ref[0, :N, :] + cnts_ref[1, :N, :])[:, :18]
    cnt = jnp.sum(cm, axis=1, keepdims=True) + 1.0
    agg = (jnp.concatenate([p_ref[0, :N, :], p_ref[1, :N, :]], axis=1)
           + jnp.concatenate([ht_ref[0], ht_ref[1]], axis=1)
           + jnp.dot(cm, etab_ref[...], preferred_element_type=jnp.float32)
           + etab_ref[12:13, :])
    out = agg / cnt
    nrm = jnp.sqrt(jnp.sum(out * out, axis=-1, keepdims=True))
    out = out / jnp.maximum(nrm, 1e-12)
    mean = jnp.mean(out, axis=0, keepdims=True)
    var = jnp.mean((out - mean) ** 2, axis=0, keepdims=True)
    out = (out - mean) / jnp.sqrt(var + 1e-5) * gamma_ref[...] + beta_ref[...]
    if relu:
        out = jnp.maximum(out, 0.0)
    out_ref[0, :, :] = out[:, :HW]
    out_ref[1, :, :] = out[:, HW:]


def _epilogue(p, ht, cnts, etab, gamma, beta, relu):
    return pl.pallas_call(
        functools.partial(_epilogue_body, relu),
        out_shape=jax.ShapeDtypeStruct((NC, N, HW), jnp.float32),
    )(p, ht, cnts, etab, gamma, beta)


def _pad_slabs(a, ng, nch, trash):
    tot = ng * nch * RPS
    pad = tot - a.shape[0]
    if trash:
        pad_vals = N + (jnp.arange(pad, dtype=jnp.int32) % (N_ACC - N))
        ext = N + (jnp.arange(ng * NBUF * RPS, dtype=jnp.int32)
                   % (N_ACC - N)).reshape(ng, NBUF, RPS)
    else:
        pad_vals = jnp.zeros((pad,), jnp.int32)
        ext = jnp.zeros((ng, NBUF, RPS), jnp.int32)
    a = jnp.concatenate([a, pad_vals])
    return jnp.concatenate([a.reshape(ng, nch, RPS), ext], axis=1)


def kernel(x, edge_index, edge_attr, x_emb1, x_emb2, e1, e2, bn_gamma, bn_beta):
    e = edge_index.shape[1]
    src = edge_index[0].astype(jnp.int32)
    dst = edge_index[1].astype(jnp.int32)
    key = (edge_attr[:, 0] * 3 + edge_attr[:, 1]).astype(jnp.int32)

    nch = -(-e // (NS * RPS))
    nch += (-nch) % NBUF
    src_p = _pad_slabs(src, NS, nch, False)
    dst_p = _pad_slabs(dst, NS, nch, True)

    nchc = -(-e // (NW * RPS))
    nchc += (-nchc) % NBUF
    key_p = _pad_slabs(key, NW, nchc, False)
    key_p = key_p + ((jnp.arange(NW) % NS) * CW)[:, None, None]
    dstc_p = _pad_slabs(dst, NW, nchc, True)

    h = (jnp.take(x_emb1, x[:, 0], axis=0)
         + jnp.take(x_emb2, x[:, 1], axis=0)).astype(jnp.float32)
    ht = jnp.stack([h[:, :HW], h[:, HW:]])

    eye = jnp.tile(jnp.eye(CW, dtype=jnp.float32), (NS, 1))
    onehot = jnp.stack([eye, eye])
    cnts = _sc_segment_sum(onehot, key_p, dstc_p, nchc, CW, partition=True)

    t_idx = jnp.arange(18) // 3
    d_idx = jnp.arange(18) % 3
    etab = e1[:, t_idx, :] + e2[:, d_idx, :]

    for l in range(L):
        p = _sc_segment_sum(ht, src_p, dst_p, nch, HW, partition=False)
        ht = _epilogue(p, ht, cnts, etab[l], bn_gamma[l][None],
                       bn_beta[l][None], l < L - 1)
    return jnp.concatenate([ht[0], ht[1]], axis=1)


# speedup vs baseline: 1.5014x; 1.5014x over previous
import functools

import jax
import jax.numpy as jnp
from jax import lax
from jax.experimental import pallas as pl
from jax.experimental.pallas import tpu as pltpu
from jax.experimental.pallas import tpu_sc as plsc

N = 10000
D = 128
L = 5

NC = 2
NS = 16
NW = NC * NS
CHUNK = 128
N_ACC = 10240
ROWS_PER_SUB = N_ACC // NS
HW = D // NC
CW = 32
NBUF = 2
SPC = 1
RPS = SPC * CHUNK


def _sc_segment_sum(table, gidx, didx, nch, width, partition):
    mesh = plsc.VectorSubcoreMesh(core_axis_name="c", subcore_axis_name="s")
    zrows = 64
    nfull = ROWS_PER_SUB // zrows

    @functools.partial(
        pl.kernel,
        mesh=mesh,
        compiler_params=pltpu.CompilerParams(use_tc_tiling_on_sc=False),
        out_type=jax.ShapeDtypeStruct((NC, N_ACC, width), jnp.float32),
        scratch_types=[
            pltpu.VMEM((nch + NBUF, RPS), jnp.int32),
            pltpu.VMEM((nch + NBUF, RPS), jnp.int32),
            pltpu.VMEM((NBUF, RPS, width), jnp.float32),
            pltpu.VMEM((zrows, width), jnp.float32),
            pltpu.VMEM_SHARED((N_ACC, width), jnp.float32),
        ] + [pltpu.SemaphoreType.DMA] * (2 * NBUF),
    )
    def k(table_hbm, gidx_hbm, didx_hbm, out_hbm,
          g_v, d_v, buf, zbuf, acc, *sems):
        semg = sems[:NBUF]
        c = lax.axis_index("c")
        s = lax.axis_index("s")
        w = c * NS + s if partition else s
        pltpu.sync_copy(gidx_hbm.at[w], g_v)
        pltpu.sync_copy(didx_hbm.at[w], d_v)

        def zrow(i, carry):
            for col in range(width // 16):
                zbuf[i, pl.ds(col * 16, 16)] = jnp.zeros((16,), jnp.float32)
            return carry
        lax.fori_loop(0, zrows, zrow, 0)

        base = s * ROWS_PER_SUB
        for kk in range(nfull):
            pltpu.sync_copy(zbuf, acc.at[pl.ds(base + kk * zrows, zrows)])
        plsc.subcore_barrier()

        tbl = table_hbm.at[c]
        for b in range(NBUF):
            pltpu.async_copy(tbl.at[g_v.at[b]], buf.at[b], semg[b])

        def body(i, carry):
            j0 = NBUF * i
            for b in range(NBUF):
                pltpu.make_async_copy(tbl.at[g_v.at[j0 + b]], buf.at[b],
                                      semg[b]).wait()
                pltpu.sync_copy(buf.at[b], acc.at[d_v.at[j0 + b]], add=True)
                pltpu.async_copy(tbl.at[g_v.at[j0 + NBUF + b]], buf.at[b],
                                 semg[b])
            return carry
        lax.fori_loop(0, nch // NBUF, body, 0)

        for b in range(NBUF):
            pltpu.make_async_copy(tbl.at[g_v.at[nch + b]], buf.at[b],
                                  semg[b]).wait()

        plsc.subcore_barrier()
        pltpu.sync_copy(acc.at[pl.ds(base, ROWS_PER_SUB)],
                        out_hbm.at[c, pl.ds(base, ROWS_PER_SUB)])

    return k(table, gidx, didx)


def _epilogue_body(relu, p_ref, ht_ref, cnts_ref, etab_ref, gamma_ref,
                   beta_ref, out_ref):
    cm = (cnts_ref[0, :N, :] + cnts_ref[1, :N, :])[:, :18]
    cnt = jnp.sum(cm, axis=1, keepdims=True) + 1.0
    agg = (jnp.concatenate([p_ref[0, :N, :], p_ref[1, :N, :]], axis=1)
           + jnp.concatenate([ht_ref[0], ht_ref[1]], axis=1)
           + jnp.dot(cm, etab_ref[...], preferred_element_type=jnp.float32)
           + etab_ref[12:13, :])
    out = agg / cnt
    nrm = jnp.sqrt(jnp.sum(out * out, axis=-1, keepdims=True))
    out = out / jnp.maximum(nrm, 1e-12)
    mean = jnp.mean(out, axis=0, keepdims=True)
    var = jnp.mean((out - mean) ** 2, axis=0, keepdims=True)
    out = (out - mean) / jnp.sqrt(var + 1e-5) * gamma_ref[...] + beta_ref[...]
    if relu:
        out = jnp.maximum(out, 0.0)
    out_ref[0, :, :] = out[:, :HW]
    out_ref[1, :, :] = out[:, HW:]


def _epilogue(p, ht, cnts, etab, gamma, beta, relu):
    return pl.pallas_call(
        functools.partial(_epilogue_body, relu),
        out_shape=jax.ShapeDtypeStruct((NC, N, HW), jnp.float32),
    )(p, ht, cnts, etab, gamma, beta)


def _pad_slabs(a, ng, nch, trash):
    tot = ng * nch * RPS
    pad = tot - a.shape[0]
    fill = N if trash else 0
    pad_vals = jnp.full((pad,), fill, jnp.int32)
    ext = jnp.full((ng, NBUF, RPS), fill, jnp.int32)
    a = jnp.concatenate([a, pad_vals])
    return jnp.concatenate([a.reshape(ng, nch, RPS), ext], axis=1)


def kernel(x, edge_index, edge_attr, x_emb1, x_emb2, e1, e2, bn_gamma, bn_beta):
    e = edge_index.shape[1]
    src = edge_index[0].astype(jnp.int32)
    dst = edge_index[1].astype(jnp.int32)
    key = (edge_attr[:, 0] * 3 + edge_attr[:, 1]).astype(jnp.int32)

    nch = -(-e // (NS * RPS))
    nch += (-nch) % NBUF
    src_p = _pad_slabs(src, NS, nch, False)
    dst_p = _pad_slabs(dst, NS, nch, True)

    nchc = -(-e // (NW * RPS))
    nchc += (-nchc) % NBUF
    key_p = _pad_slabs(key, NW, nchc, False)
    key_p = key_p + ((jnp.arange(NW) % NS) * CW)[:, None, None]
    dstc_p = _pad_slabs(dst, NW, nchc, True)

    h = (jnp.take(x_emb1, x[:, 0], axis=0)
         + jnp.take(x_emb2, x[:, 1], axis=0)).astype(jnp.float32)
    ht = jnp.stack([h[:, :HW], h[:, HW:]])

    eye = jnp.tile(jnp.eye(CW, dtype=jnp.float32), (NS, 1))
    onehot = jnp.stack([eye, eye])
    cnts = _sc_segment_sum(onehot, key_p, dstc_p, nchc, CW, partition=True)

    t_idx = jnp.arange(18) // 3
    d_idx = jnp.arange(18) % 3
    etab = e1[:, t_idx, :] + e2[:, d_idx, :]

    for l in range(L):
        p = _sc_segment_sum(ht, src_p, dst_p, nch, HW, partition=False)
        ht = _epilogue(p, ht, cnts, etab[l], bn_gamma[l][None],
                       bn_beta[l][None], l < L - 1)
    return jnp.concatenate([ht[0], ht[1]], axis=1)
